# baseline (speedup 1.0000x reference)
import jax
import jax.numpy as jnp
from jax.experimental import pallas as pl
from jax.experimental.pallas import tpu as pltpu

_A, _B, _E, _M, _H = 4, 4, 8, 128, 1024
_I = 1024
_G = _A * _B
_S = 2048
_GP = 4
_MM = _GP * _M


def _fused_expert_body(x_ref, guw_ref, gub_ref, dw_ref, db_ref, o_ref,
                       psel_s, guw_s, dw_s, gub_s):
    @pl.when((pl.program_id(0) == 0) & (pl.program_id(1) == 0))
    def _build_psel():
        n = jax.lax.broadcasted_iota(jnp.int32, (2 * _I, 2 * _I), 0)
        i = jax.lax.broadcasted_iota(jnp.int32, (2 * _I, 2 * _I), 1)
        src = jnp.where(i < _I, 2 * i, 2 * (i - _I) + 1)
        psel_s[...] = (n == src).astype(jnp.bfloat16)

    @pl.when(pl.program_id(1) == 0)
    def _prep():
        w = guw_ref[0].astype(jnp.bfloat16)
        guw_s[...] = jnp.dot(w, psel_s[...],
                             preferred_element_type=jnp.float32).astype(jnp.bfloat16)
        dw_s[...] = dw_ref[0].astype(jnp.bfloat16)
        b = gub_ref[0].astype(jnp.bfloat16)
        gub_s[...] = jnp.dot(b, psel_s[...], preferred_element_type=jnp.float32)

    x = x_ref[0, :, 0].reshape(_MM, _H).astype(jnp.bfloat16)
    gu = jnp.dot(x, guw_s[...], preferred_element_type=jnp.float32)
    gu = gu + gub_s[...]
    gate = gu[:, :_I]
    up = gu[:, _I:]
    act = (gate * jax.nn.sigmoid(gate)) * up
    act = act.astype(jnp.bfloat16)
    out = jnp.dot(act, dw_s[...], preferred_element_type=jnp.float32)
    o_ref[...] = out + db_ref[0]


def kernel(dispatched, gate_up_proj, gate_up_proj_bias, down_proj, down_proj_bias, sparsity_remap):
    del sparsity_remap

    gub = gate_up_proj_bias.reshape(_E, 1, 2 * _I)
    db = down_proj_bias.reshape(_E, 1, _H)

    out2d = pl.pallas_call(
        _fused_expert_body,
        grid=(_E, _G // _GP),
        in_specs=[
            pl.BlockSpec((1, _B, 1, _M, _H), lambda e, g: (g, 0, e, 0, 0)),
            pl.BlockSpec((1, _H, 2 * _I), lambda e, g: (e, 0, 0)),
            pl.BlockSpec((1, 1, 2 * _I), lambda e, g: (e, 0, 0)),
            pl.BlockSpec((1, _I, _H), lambda e, g: (e, 0, 0)),
            pl.BlockSpec((1, 1, _H), lambda e, g: (e, 0, 0)),
        ],
        out_specs=pl.BlockSpec((_MM, _H), lambda e, g: (g, e)),
        out_shape=jax.ShapeDtypeStruct((_S, _E * _H), jnp.float32),
        scratch_shapes=[
            pltpu.VMEM((2 * _I, 2 * _I), jnp.bfloat16),
            pltpu.VMEM((_H, 2 * _I), jnp.bfloat16),
            pltpu.VMEM((_I, _H), jnp.bfloat16),
            pltpu.VMEM((1, 2 * _I), jnp.float32),
        ],
        compiler_params=pltpu.CompilerParams(
            dimension_semantics=("arbitrary", "arbitrary"),
        ),
    )(dispatched, gate_up_proj, gub, down_proj, db)

    return out2d.reshape(1, _S, _E, _H)


# speedup vs baseline: 1.2359x; 1.2359x over previous
import jax
import jax.numpy as jnp
from jax.experimental import pallas as pl
from jax.experimental.pallas import tpu as pltpu

_A, _B, _E, _M, _H = 4, 4, 8, 128, 1024
_I = 1024
_G = _A * _B
_S = 2048
_GP = 4
_MM = _GP * _M
_NG = _G // _GP


def _fused_expert_body(x_ref, guw_ref, gub_ref, dw_ref, db_ref, o_hbm,
                       psel_s, guw_s, dw_s, gub_s, ovmem, osem):
    e = pl.program_id(0)
    g = pl.program_id(1)
    step = e * _NG + g
    slot = jax.lax.rem(step, 2)

    @pl.when((e == 0) & (g == 0))
    def _build_psel():
        n = jax.lax.broadcasted_iota(jnp.int32, (2 * _I, 2 * _I), 0)
        i = jax.lax.broadcasted_iota(jnp.int32, (2 * _I, 2 * _I), 1)
        src = jnp.where(i < _I, 2 * i, 2 * (i - _I) + 1)
        psel_s[...] = (n == src).astype(jnp.bfloat16)

    @pl.when(g == 0)
    def _prep():
        w = guw_ref[0].astype(jnp.bfloat16)
        guw_s[...] = jnp.dot(w, psel_s[...],
                             preferred_element_type=jnp.float32).astype(jnp.bfloat16)
        dw_s[...] = dw_ref[0].astype(jnp.bfloat16)
        b = gub_ref[0].astype(jnp.bfloat16)
        gub_s[...] = jnp.dot(b, psel_s[...], preferred_element_type=jnp.float32)

    x = x_ref[0, :, 0].reshape(_MM, _H).astype(jnp.bfloat16)
    gu = jnp.dot(x, guw_s[...], preferred_element_type=jnp.float32)
    gu = gu + gub_s[...]
    gate = gu[:, :_I]
    up = gu[:, _I:]
    act = (gate * jax.nn.sigmoid(gate)) * up
    act = act.astype(jnp.bfloat16)
    out = jnp.dot(act, dw_s[...], preferred_element_type=jnp.float32)

    def _copy(s, ee, gg):
        return pltpu.make_async_copy(
            ovmem.at[s],
            o_hbm.at[0, pl.ds(gg * _MM, _MM), ee, :],
            osem.at[s])

    @pl.when(step >= 2)
    def _wait_prev():
        _copy(slot, e, g).wait()

    ovmem[slot] = out + db_ref[0]
    _copy(slot, e, g).start()

    @pl.when(step == _E * _NG - 1)
    def _drain():
        _copy(1 - slot, e, g).wait()
        _copy(slot, e, g).wait()


def kernel(dispatched, gate_up_proj, gate_up_proj_bias, down_proj, down_proj_bias, sparsity_remap):
    del sparsity_remap

    gub = gate_up_proj_bias.reshape(_E, 1, 2 * _I)
    db = down_proj_bias.reshape(_E, 1, _H)

    return pl.pallas_call(
        _fused_expert_body,
        grid=(_E, _NG),
        in_specs=[
            pl.BlockSpec((1, _B, 1, _M, _H), lambda e, g: (g, 0, e, 0, 0)),
            pl.BlockSpec((1, _H, 2 * _I), lambda e, g: (e, 0, 0)),
            pl.BlockSpec((1, 1, 2 * _I), lambda e, g: (e, 0, 0)),
            pl.BlockSpec((1, _I, _H), lambda e, g: (e, 0, 0)),
            pl.BlockSpec((1, 1, _H), lambda e, g: (e, 0, 0)),
        ],
        out_specs=pl.BlockSpec(memory_space=pl.ANY),
        out_shape=jax.ShapeDtypeStruct((1, _S, _E, _H), jnp.float32),
        scratch_shapes=[
            pltpu.VMEM((2 * _I, 2 * _I), jnp.bfloat16),
            pltpu.VMEM((_H, 2 * _I), jnp.bfloat16),
            pltpu.VMEM((_I, _H), jnp.bfloat16),
            pltpu.VMEM((1, 2 * _I), jnp.float32),
            pltpu.VMEM((2, _MM, _H), jnp.float32),
            pltpu.SemaphoreType.DMA((2,)),
        ],
        compiler_params=pltpu.CompilerParams(
            dimension_semantics=("arbitrary", "arbitrary"),
        ),
    )(dispatched, gate_up_proj, gub, down_proj, db)


# speedup vs baseline: 1.7309x; 1.4005x over previous
import jax
import jax.numpy as jnp
from jax.experimental import pallas as pl
from jax.experimental.pallas import tpu as pltpu

_A, _B, _E, _M, _H = 4, 4, 8, 128, 1024
_I = 1024
_G = _A * _B
_S = 2048
_GP = 4
_MM = _GP * _M
_NG = _G // _GP
_BK = 256


def _fused_expert_body(x_ref, guw_ref, gub_ref, dw_ref, db_ref, o_hbm,
                       psel_s, guw_s, dw_s, gub_s, ovmem, osem):
    e = pl.program_id(0)
    g = pl.program_id(1)
    step = e * _NG + g
    slot = jax.lax.rem(step, 2)

    @pl.when((e == 0) & (g == 0))
    def _build_psel():
        n = jax.lax.broadcasted_iota(jnp.int32, (_BK, _BK), 0)
        j = jax.lax.broadcasted_iota(jnp.int32, (_BK, _BK), 1)
        src = jnp.where(j < _BK // 2, 2 * j, 2 * (j - _BK // 2) + 1)
        psel_s[...] = (n == src).astype(jnp.bfloat16)

    @pl.when(g == 0)
    def _prep():
        p256 = psel_s[...]
        w = guw_ref[0].astype(jnp.bfloat16)
        b = gub_ref[0].astype(jnp.bfloat16)
        wg, wu, bg, bu = [], [], [], []
        for blk in range(2 * _I // _BK):
            dw_blk = jnp.dot(w[:, blk * _BK:(blk + 1) * _BK], p256,
                             preferred_element_type=jnp.float32).astype(jnp.bfloat16)
            wg.append(dw_blk[:, :_BK // 2])
            wu.append(dw_blk[:, _BK // 2:])
            db_blk = jnp.dot(b[:, blk * _BK:(blk + 1) * _BK], p256,
                             preferred_element_type=jnp.float32)
            bg.append(db_blk[:, :_BK // 2])
            bu.append(db_blk[:, _BK // 2:])
        guw_s[...] = jnp.concatenate(wg + wu, axis=1)
        gub_s[...] = jnp.concatenate(bg + bu, axis=1)
        dw_s[...] = dw_ref[0].astype(jnp.bfloat16)

    x = x_ref[0, :, 0].reshape(_MM, _H).astype(jnp.bfloat16)
    gu = jnp.dot(x, guw_s[...], preferred_element_type=jnp.float32)
    gu = gu + gub_s[...]
    gate = gu[:, :_I]
    up = gu[:, _I:]
    act = (gate * jax.nn.sigmoid(gate)) * up
    act = act.astype(jnp.bfloat16)
    out = jnp.dot(act, dw_s[...], preferred_element_type=jnp.float32)

    def _copy(s, ee, gg):
        return pltpu.make_async_copy(
            ovmem.at[s],
            o_hbm.at[0, pl.ds(gg * _MM, _MM), ee, :],
            osem.at[s])

    @pl.when(step >= 2)
    def _wait_prev():
        _copy(slot, e, g).wait()

    ovmem[slot] = out + db_ref[0]
    _copy(slot, e, g).start()

    @pl.when(step == _E * _NG - 1)
    def _drain():
        _copy(1 - slot, e, g).wait()
        _copy(slot, e, g).wait()


def kernel(dispatched, gate_up_proj, gate_up_proj_bias, down_proj, down_proj_bias, sparsity_remap):
    del sparsity_remap

    gub = gate_up_proj_bias.reshape(_E, 1, 2 * _I)
    db = down_proj_bias.reshape(_E, 1, _H)

    return pl.pallas_call(
        _fused_expert_body,
        grid=(_E, _NG),
        in_specs=[
            pl.BlockSpec((1, _B, 1, _M, _H), lambda e, g: (g, 0, e, 0, 0)),
            pl.BlockSpec((1, _H, 2 * _I), lambda e, g: (e, 0, 0)),
            pl.BlockSpec((1, 1, 2 * _I), lambda e, g: (e, 0, 0)),
            pl.BlockSpec((1, _I, _H), lambda e, g: (e, 0, 0)),
            pl.BlockSpec((1, 1, _H), lambda e, g: (e, 0, 0)),
        ],
        out_specs=pl.BlockSpec(memory_space=pl.ANY),
        out_shape=jax.ShapeDtypeStruct((1, _S, _E, _H), jnp.float32),
        scratch_shapes=[
            pltpu.VMEM((_BK, _BK), jnp.bfloat16),
            pltpu.VMEM((_H, 2 * _I), jnp.bfloat16),
            pltpu.VMEM((_I, _H), jnp.bfloat16),
            pltpu.VMEM((1, 2 * _I), jnp.float32),
            pltpu.VMEM((2, _MM, _H), jnp.float32),
            pltpu.SemaphoreType.DMA((2,)),
        ],
        compiler_params=pltpu.CompilerParams(
            dimension_semantics=("arbitrary", "arbitrary"),
        ),
    )(dispatched, gate_up_proj, gub, down_proj, db)
